# baseline (speedup 1.0000x reference)
import functools

import jax
import jax.numpy as jnp
from jax import lax
from jax.experimental import pallas as pl
from jax.experimental.pallas import tpu as pltpu
from jax.experimental.pallas import tpu_sc as plsc

N = 10000
D = 128
E = 320000
CHUNK = 128
NCHUNKS = 2560
EPAD = NCHUNKS * CHUNK - E
DUMMY = 10000
NC, NS = 2, 16
NW = NC * NS
CPT = NCHUNKS // NW
PASSES = 2
CPP = CPT // PASSES
NPAD = 10240
RPS = NPAD // NS
ZROWS = 16
LANES = 16


def _matmul_body(h_ref, w_ref, norm_ref, x_ref):
    x_ref[...] = (
        jnp.dot(h_ref[...], w_ref[...], preferred_element_type=jnp.float32)
        * norm_ref[...]
    )


def _xw_norm(h, W, norm):
    mb = 2000
    return pl.pallas_call(
        _matmul_body,
        grid=(N // mb,),
        in_specs=[
            pl.BlockSpec((mb, D), lambda i: (i, 0)),
            pl.BlockSpec((D, D), lambda i: (0, 0)),
            pl.BlockSpec((mb, 1), lambda i: (i, 0)),
        ],
        out_specs=pl.BlockSpec((mb, D), lambda i: (i, 0)),
        out_shape=jax.ShapeDtypeStruct((N, D), jnp.float32),
    )(h, W, norm)


def _sc_body(x_hbm, src_hbm, dst_hbm, out_hbm,
             sidx, didx, rows0, rows1, zbuf, acc, sem0, sem1):
    c = lax.axis_index("c")
    s = lax.axis_index("s")
    wid = c * NS + s

    zeros = jnp.zeros((LANES,), jnp.float32)

    def zrow(r, carry):
        for k in range(D // LANES):
            zbuf[r, pl.ds(k * LANES, LANES)] = zeros
        return carry

    lax.fori_loop(0, ZROWS, zrow, 0)

    def zcopy(i, carry):
        pltpu.sync_copy(zbuf, acc.at[pl.ds(s * RPS + i * ZROWS, ZROWS)])
        return carry

    lax.fori_loop(0, RPS // ZROWS, zcopy, 0)
    plsc.subcore_barrier()

    rows = (rows0, rows1)
    sems = (sem0, sem1)

    for p in range(PASSES):
        base = wid * CPT + p * CPP
        pltpu.sync_copy(src_hbm.at[pl.ds(base, CPP)], sidx)
        pltpu.sync_copy(dst_hbm.at[pl.ds(base, CPP)], didx)

        pltpu.async_copy(x_hbm.at[sidx.at[0]], rows0, sem0)
        pltpu.async_copy(x_hbm.at[sidx.at[1]], rows1, sem1)

        def body(g, carry):
            for b in range(2):
                j = g * 2 + b
                pltpu.make_async_copy(
                    x_hbm.at[sidx.at[j]], rows[b], sems[b]).wait()
                pltpu.sync_copy(rows[b], acc.at[didx.at[j]], add=True)

                @pl.when(j + 2 < CPP)
                def _():
                    pltpu.async_copy(x_hbm.at[sidx.at[j + 2]], rows[b], sems[b])

            return carry

        lax.fori_loop(0, CPP // 2, body, 0)

    plsc.subcore_barrier()
    pltpu.sync_copy(
        acc.at[pl.ds(s * RPS, RPS)],
        out_hbm.at[c].at[pl.ds(s * RPS, RPS)],
    )


def _sc_segment_sum(x, src_chunks, dst_chunks):
    mesh = plsc.VectorSubcoreMesh(core_axis_name="c", subcore_axis_name="s")
    f = functools.partial(
        pl.kernel,
        out_type=jax.ShapeDtypeStruct((NC, NPAD, D), jnp.float32),
        mesh=mesh,
        scratch_types=[
            pltpu.VMEM((CPP, CHUNK), jnp.int32),
            pltpu.VMEM((CPP, CHUNK), jnp.int32),
            pltpu.VMEM((CHUNK, D), jnp.float32),
            pltpu.VMEM((CHUNK, D), jnp.float32),
            pltpu.VMEM((ZROWS, D), jnp.float32),
            pltpu.VMEM_SHARED((NPAD, D), jnp.float32),
            pltpu.SemaphoreType.DMA,
            pltpu.SemaphoreType.DMA,
        ],
    )(_sc_body)
    return f(x, src_chunks, dst_chunks)


def _combine_body(p_ref, norm_ref, b_ref, o_ref):
    o_ref[...] = jnp.maximum(
        (p_ref[0] + p_ref[1]) * norm_ref[...] + b_ref[...], 0.0
    )


def _combine(partials, norm, b2d):
    mb = 2000
    return pl.pallas_call(
        _combine_body,
        grid=(N // mb,),
        in_specs=[
            pl.BlockSpec((NC, mb, D), lambda i: (0, i, 0)),
            pl.BlockSpec((mb, 1), lambda i: (i, 0)),
            pl.BlockSpec((1, D), lambda i: (0, 0)),
        ],
        out_specs=pl.BlockSpec((mb, D), lambda i: (i, 0)),
        out_shape=jax.ShapeDtypeStruct((N, D), jnp.float32),
    )(partials, norm, b2d)


def kernel(h, edge_index, norm, W, b):
    x = _xw_norm(h, W, norm)
    src_chunks = jnp.concatenate(
        [edge_index[0], jnp.zeros((EPAD,), jnp.int32)]).reshape(NCHUNKS, CHUNK)
    dummy_dst = DUMMY + (jnp.arange(EPAD, dtype=jnp.int32) % 128)
    dst_chunks = jnp.concatenate(
        [edge_index[1], dummy_dst]).reshape(NCHUNKS, CHUNK)
    partials = _sc_segment_sum(x, src_chunks, dst_chunks)
    return _combine(partials, norm, b.reshape(1, D))


# speedup vs baseline: 3.1991x; 3.1991x over previous
import functools

import jax
import jax.numpy as jnp
from jax import lax
from jax.experimental import pallas as pl
from jax.experimental.pallas import tpu as pltpu
from jax.experimental.pallas import tpu_sc as plsc

N = 10000
D = 128
E = 320000
CHUNK = 128
NCHUNKS = 2560
EPAD = NCHUNKS * CHUNK - E
DUMMY = 10000
NC, NS = 2, 16
NW = NC * NS
CPT = NCHUNKS // NW
PASSES = 2
CPP = CPT // PASSES
NPAD = 10240
RPS = NPAD // NS
ZROWS = 16
LANES = 16


def _matmul_body(h_ref, w_ref, norm_ref, x_ref):
    x_ref[...] = (
        jnp.dot(h_ref[...], w_ref[...], preferred_element_type=jnp.float32)
        * norm_ref[...]
    )


def _xw_norm(h, W, norm):
    mb = 2000
    return pl.pallas_call(
        _matmul_body,
        grid=(N // mb,),
        in_specs=[
            pl.BlockSpec((mb, D), lambda i: (i, 0)),
            pl.BlockSpec((D, D), lambda i: (0, 0)),
            pl.BlockSpec((mb, 1), lambda i: (i, 0)),
        ],
        out_specs=pl.BlockSpec((mb, D), lambda i: (i, 0)),
        out_shape=jax.ShapeDtypeStruct((N, D), jnp.float32),
    )(h, W, norm)


def _sc_body(x_hbm, src_hbm, dst_hbm, out_hbm,
             sidx, didx, rows0, rows1, zbuf, acc, sem0, sem1):
    c = lax.axis_index("c")
    s = lax.axis_index("s")
    wid = c * NS + s

    zeros = jnp.zeros((LANES,), jnp.float32)

    def zrow(r, carry):
        for k in range(D // LANES):
            zbuf[r, pl.ds(k * LANES, LANES)] = zeros
        return carry

    lax.fori_loop(0, ZROWS, zrow, 0)

    def zcopy(i, carry):
        pltpu.sync_copy(zbuf, acc.at[pl.ds(s * RPS + i * ZROWS, ZROWS)])
        return carry

    lax.fori_loop(0, RPS // ZROWS, zcopy, 0)
    plsc.subcore_barrier()

    rows = (rows0, rows1)
    sems = (sem0, sem1)

    for p in range(PASSES):
        base = wid * CPT + p * CPP
        pltpu.sync_copy(src_hbm.at[pl.ds(base, CPP)], sidx)
        pltpu.sync_copy(dst_hbm.at[pl.ds(base, CPP)], didx)

        pltpu.async_copy(x_hbm.at[sidx.at[0]], rows0, sem0)
        pltpu.async_copy(x_hbm.at[sidx.at[1]], rows1, sem1)

        def body(g, carry):
            for b in range(2):
                j = g * 2 + b
                pltpu.make_async_copy(
                    x_hbm.at[sidx.at[j]], rows[b], sems[b]).wait()
                pltpu.sync_copy(rows[b], acc.at[didx.at[j]], add=True)

                @pl.when(j + 2 < CPP)
                def _():
                    pltpu.async_copy(x_hbm.at[sidx.at[j + 2]], rows[b], sems[b])

            return carry

        lax.fori_loop(0, CPP // 2, body, 0)

    plsc.subcore_barrier()
    pltpu.sync_copy(
        acc.at[pl.ds(s * RPS, RPS)],
        out_hbm.at[c].at[pl.ds(s * RPS, RPS)],
    )


def _sc_segment_sum(x, src_chunks, dst_chunks):
    mesh = plsc.VectorSubcoreMesh(core_axis_name="c", subcore_axis_name="s")
    f = functools.partial(
        pl.kernel,
        out_type=jax.ShapeDtypeStruct((NC, NPAD, D), jnp.float32),
        mesh=mesh,
        scratch_types=[
            pltpu.VMEM((CPP, CHUNK), jnp.int32),
            pltpu.VMEM((CPP, CHUNK), jnp.int32),
            pltpu.VMEM((CHUNK, D), jnp.float32),
            pltpu.VMEM((CHUNK, D), jnp.float32),
            pltpu.VMEM((ZROWS, D), jnp.float32),
            pltpu.VMEM_SHARED((NPAD, D), jnp.float32),
            pltpu.SemaphoreType.DMA,
            pltpu.SemaphoreType.DMA,
        ],
    )(_sc_body)
    return f(x, src_chunks, dst_chunks)


def _combine_body(p_ref, norm_ref, b_ref, o_ref):
    o_ref[...] = jnp.maximum(
        (p_ref[0] + p_ref[1]) * norm_ref[...] + b_ref[...], 0.0
    )


def _combine(partials, norm, b2d):
    mb = 2000
    return pl.pallas_call(
        _combine_body,
        grid=(N // mb,),
        in_specs=[
            pl.BlockSpec((NC, mb, D), lambda i: (0, i, 0)),
            pl.BlockSpec((mb, 1), lambda i: (i, 0)),
            pl.BlockSpec((1, D), lambda i: (0, 0)),
        ],
        out_specs=pl.BlockSpec((mb, D), lambda i: (i, 0)),
        out_shape=jax.ShapeDtypeStruct((N, D), jnp.float32),
    )(partials, norm, b2d)


def kernel(h, edge_index, norm, W, b):
    x = _xw_norm(h, W, norm)
    dummy_src = jnp.arange(EPAD, dtype=jnp.int32) % N
    src_chunks = jnp.concatenate(
        [edge_index[0], dummy_src]).reshape(NCHUNKS, CHUNK)
    dummy_dst = DUMMY + (jnp.arange(EPAD, dtype=jnp.int32) % 128)
    dst_chunks = jnp.concatenate(
        [edge_index[1], dummy_dst]).reshape(NCHUNKS, CHUNK)
    partials = _sc_segment_sum(x, src_chunks, dst_chunks)
    return _combine(partials, norm, b.reshape(1, D))


# speedup vs baseline: 3.2362x; 1.0116x over previous
import functools

import jax
import jax.numpy as jnp
from jax import lax
from jax.experimental import pallas as pl
from jax.experimental.pallas import tpu as pltpu
from jax.experimental.pallas import tpu_sc as plsc

N = 10000
D = 128
E = 320000
CHUNK = 128
NCHUNKS = 2560
EPAD = NCHUNKS * CHUNK - E
NREAL = E // CHUNK
NC, NS = 2, 16
NW = NC * NS
CPT = NCHUNKS // NW
PASSES = 2
CPP = CPT // PASSES
NPAD = 10240
RPS = NPAD // NS
ZROWS = 16
LANES = 16


def _matmul_body(h_ref, w_ref, norm_ref, x_ref):
    x_ref[...] = (
        jnp.dot(h_ref[...], w_ref[...], preferred_element_type=jnp.float32)
        * norm_ref[...]
    )


def _xw_norm(h, W, norm):
    mb = 2000
    return pl.pallas_call(
        _matmul_body,
        grid=(N // mb,),
        in_specs=[
            pl.BlockSpec((mb, D), lambda i: (i, 0)),
            pl.BlockSpec((D, D), lambda i: (0, 0)),
            pl.BlockSpec((mb, 1), lambda i: (i, 0)),
        ],
        out_specs=pl.BlockSpec((mb, D), lambda i: (i, 0)),
        out_shape=jax.ShapeDtypeStruct((N, D), jnp.float32),
    )(h, W, norm)


def _sc_body(x_hbm, src_hbm, dst_hbm, out_hbm,
             sidx, didx, rows0, rows1, zbuf, acc, sem0, sem1):
    c = lax.axis_index("c")
    s = lax.axis_index("s")
    wid = c * NS + s

    zeros = jnp.zeros((LANES,), jnp.float32)

    def zrow(r, carry):
        for k in range(D // LANES):
            zbuf[r, pl.ds(k * LANES, LANES)] = zeros
        return carry

    lax.fori_loop(0, ZROWS, zrow, 0)

    def zcopy(i, carry):
        pltpu.sync_copy(zbuf, acc.at[pl.ds(s * RPS + i * ZROWS, ZROWS)])
        return carry

    rows = (rows0, rows1)
    sems = (sem0, sem1)
    ebase = wid * CPT

    def prime(base):
        for b in range(2):
            @pl.when(base + b < NREAL)
            def _():
                pltpu.async_copy(x_hbm.at[sidx.at[b]], rows[b], sems[b])

    pltpu.sync_copy(src_hbm.at[pl.ds(ebase, CPP)], sidx)
    pltpu.sync_copy(dst_hbm.at[pl.ds(ebase, CPP)], didx)
    prime(ebase)

    lax.fori_loop(0, RPS // ZROWS, zcopy, 0)
    plsc.subcore_barrier()

    for p in range(PASSES):
        base = ebase + p * CPP
        if p > 0:
            pltpu.sync_copy(src_hbm.at[pl.ds(base, CPP)], sidx)
            pltpu.sync_copy(dst_hbm.at[pl.ds(base, CPP)], didx)
            prime(base)

        def body(g, carry):
            for b in range(2):
                j = g * 2 + b

                @pl.when(base + j < NREAL)
                def _():
                    pltpu.make_async_copy(
                        x_hbm.at[sidx.at[j]], rows[b], sems[b]).wait()
                    pltpu.sync_copy(rows[b], acc.at[didx.at[j]], add=True)

                    @pl.when((j + 2 < CPP) & (base + j + 2 < NREAL))
                    def _():
                        pltpu.async_copy(
                            x_hbm.at[sidx.at[j + 2]], rows[b], sems[b])

            return carry

        lax.fori_loop(0, CPP // 2, body, 0)

    plsc.subcore_barrier()
    pltpu.sync_copy(
        acc.at[pl.ds(s * RPS, RPS)],
        out_hbm.at[c].at[pl.ds(s * RPS, RPS)],
    )


def _sc_segment_sum(x, src_chunks, dst_chunks):
    mesh = plsc.VectorSubcoreMesh(core_axis_name="c", subcore_axis_name="s")
    f = functools.partial(
        pl.kernel,
        out_type=jax.ShapeDtypeStruct((NC, NPAD, D), jnp.float32),
        mesh=mesh,
        scratch_types=[
            pltpu.VMEM((CPP, CHUNK), jnp.int32),
            pltpu.VMEM((CPP, CHUNK), jnp.int32),
            pltpu.VMEM((CHUNK, D), jnp.float32),
            pltpu.VMEM((CHUNK, D), jnp.float32),
            pltpu.VMEM((ZROWS, D), jnp.float32),
            pltpu.VMEM_SHARED((NPAD, D), jnp.float32),
            pltpu.SemaphoreType.DMA,
            pltpu.SemaphoreType.DMA,
        ],
    )(_sc_body)
    return f(x, src_chunks, dst_chunks)


def _combine_body(p_ref, norm_ref, b_ref, o_ref):
    o_ref[...] = jnp.maximum(
        (p_ref[0] + p_ref[1]) * norm_ref[...] + b_ref[...], 0.0
    )


def _combine(partials, norm, b2d):
    mb = 2000
    return pl.pallas_call(
        _combine_body,
        grid=(N // mb,),
        in_specs=[
            pl.BlockSpec((NC, mb, D), lambda i: (0, i, 0)),
            pl.BlockSpec((mb, 1), lambda i: (i, 0)),
            pl.BlockSpec((1, D), lambda i: (0, 0)),
        ],
        out_specs=pl.BlockSpec((mb, D), lambda i: (i, 0)),
        out_shape=jax.ShapeDtypeStruct((N, D), jnp.float32),
    )(partials, norm, b2d)


def kernel(h, edge_index, norm, W, b):
    x = _xw_norm(h, W, norm)
    pad = jnp.zeros((EPAD,), jnp.int32)
    src_chunks = jnp.concatenate([edge_index[0], pad]).reshape(NCHUNKS, CHUNK)
    dst_chunks = jnp.concatenate([edge_index[1], pad]).reshape(NCHUNKS, CHUNK)
    partials = _sc_segment_sum(x, src_chunks, dst_chunks)
    return _combine(partials, norm, b.reshape(1, D))


# speedup vs baseline: 3.2363x; 1.0000x over previous
import functools

import jax
import jax.numpy as jnp
from jax import lax
from jax.experimental import pallas as pl
from jax.experimental.pallas import tpu as pltpu
from jax.experimental.pallas import tpu_sc as plsc

N = 10000
D = 128
E = 320000
CHUNK = 128
NCHUNKS = 2560
EPAD = NCHUNKS * CHUNK - E
NREAL = E // CHUNK
NC, NS = 2, 16
NW = NC * NS
CPT = NCHUNKS // NW
PASSES = 2
CPP = CPT // PASSES
NPAD = 10240
RPS = NPAD // NS
ZROWS = 16
LANES = 16


def _matmul_body(h_ref, w_ref, norm_ref, x_ref):
    x_ref[...] = (
        jnp.dot(h_ref[...], w_ref[...], preferred_element_type=jnp.float32)
        * norm_ref[...]
    )


def _xw_norm(h, W, norm):
    mb = 2000
    return pl.pallas_call(
        _matmul_body,
        grid=(N // mb,),
        in_specs=[
            pl.BlockSpec((mb, D), lambda i: (i, 0)),
            pl.BlockSpec((D, D), lambda i: (0, 0)),
            pl.BlockSpec((mb, 1), lambda i: (i, 0)),
        ],
        out_specs=pl.BlockSpec((mb, D), lambda i: (i, 0)),
        out_shape=jax.ShapeDtypeStruct((N, D), jnp.float32),
    )(h, W, norm)


def _sc_body(x_hbm, edge_hbm, dst_hbm, out_hbm,
             sidx, didx, rows0, rows1, zbuf, acc, sem0, sem1):
    c = lax.axis_index("c")
    s = lax.axis_index("s")
    wid = c * NS + s

    zeros = jnp.zeros((LANES,), jnp.float32)

    def zrow(r, carry):
        for k in range(D // LANES):
            zbuf[r, pl.ds(k * LANES, LANES)] = zeros
        return carry

    lax.fori_loop(0, ZROWS, zrow, 0)

    def zcopy(i, carry):
        pltpu.sync_copy(zbuf, acc.at[pl.ds(s * RPS + i * ZROWS, ZROWS)])
        return carry

    rows = (rows0, rows1)
    sems = (sem0, sem1)
    ebase = wid * CPT

    def sidx_at(j):
        return sidx.at[pl.ds(j * CHUNK, CHUNK)]

    def stage_src(base):
        sbase = jnp.minimum(base, NREAL - CPP)
        pltpu.sync_copy(
            edge_hbm.at[0, pl.ds(sbase * CHUNK, CPP * CHUNK)], sidx)
        return base - sbase

    def prime(base, soff):
        for b in range(2):
            @pl.when(base + b < NREAL)
            def _():
                pltpu.async_copy(x_hbm.at[sidx_at(b + soff)], rows[b], sems[b])

    soff = stage_src(ebase)
    pltpu.sync_copy(dst_hbm.at[pl.ds(ebase, CPP)], didx)
    prime(ebase, soff)

    lax.fori_loop(0, RPS // ZROWS, zcopy, 0)
    plsc.subcore_barrier()

    for p in range(PASSES):
        base = ebase + p * CPP
        if p > 0:
            soff = stage_src(base)
            pltpu.sync_copy(dst_hbm.at[pl.ds(base, CPP)], didx)
            prime(base, soff)

        def body(g, carry):
            for b in range(2):
                j = g * 2 + b

                @pl.when(base + j < NREAL)
                def _():
                    pltpu.make_async_copy(
                        x_hbm.at[sidx_at(j + soff)], rows[b], sems[b]).wait()
                    pltpu.sync_copy(rows[b], acc.at[didx.at[j]], add=True)

                    @pl.when((j + 2 < CPP) & (base + j + 2 < NREAL))
                    def _():
                        pltpu.async_copy(
                            x_hbm.at[sidx_at(j + 2 + soff)], rows[b], sems[b])

            return carry

        lax.fori_loop(0, CPP // 2, body, 0)

    plsc.subcore_barrier()
    pltpu.sync_copy(
        acc.at[pl.ds(s * RPS, RPS)],
        out_hbm.at[c].at[pl.ds(s * RPS, RPS)],
    )


def _sc_segment_sum(x, edge_index, dst_chunks):
    mesh = plsc.VectorSubcoreMesh(core_axis_name="c", subcore_axis_name="s")
    f = functools.partial(
        pl.kernel,
        out_type=jax.ShapeDtypeStruct((NC, NPAD, D), jnp.float32),
        mesh=mesh,
        scratch_types=[
            pltpu.VMEM((CPP * CHUNK,), jnp.int32),
            pltpu.VMEM((CPP, CHUNK), jnp.int32),
            pltpu.VMEM((CHUNK, D), jnp.float32),
            pltpu.VMEM((CHUNK, D), jnp.float32),
            pltpu.VMEM((ZROWS, D), jnp.float32),
            pltpu.VMEM_SHARED((NPAD, D), jnp.float32),
            pltpu.SemaphoreType.DMA,
            pltpu.SemaphoreType.DMA,
        ],
    )(_sc_body)
    return f(x, edge_index, dst_chunks)


def _combine_body(p_ref, norm_ref, b_ref, o_ref):
    o_ref[...] = jnp.maximum(
        (p_ref[0] + p_ref[1]) * norm_ref[...] + b_ref[...], 0.0
    )


def _combine(partials, norm, b2d):
    mb = 2000
    return pl.pallas_call(
        _combine_body,
        grid=(N // mb,),
        in_specs=[
            pl.BlockSpec((NC, mb, D), lambda i: (0, i, 0)),
            pl.BlockSpec((mb, 1), lambda i: (i, 0)),
            pl.BlockSpec((1, D), lambda i: (0, 0)),
        ],
        out_specs=pl.BlockSpec((mb, D), lambda i: (i, 0)),
        out_shape=jax.ShapeDtypeStruct((N, D), jnp.float32),
    )(partials, norm, b2d)


def kernel(h, edge_index, norm, W, b):
    x = _xw_norm(h, W, norm)
    pad = jnp.zeros((EPAD,), jnp.int32)
    dst_chunks = jnp.concatenate([edge_index[1], pad]).reshape(NCHUNKS, CHUNK)
    partials = _sc_segment_sum(x, edge_index, dst_chunks)
    return _combine(partials, norm, b.reshape(1, D))


# speedup vs baseline: 3.5383x; 1.0933x over previous
import functools

import jax
import jax.numpy as jnp
from jax import lax
from jax.experimental import pallas as pl
from jax.experimental.pallas import tpu as pltpu
from jax.experimental.pallas import tpu_sc as plsc

N = 10000
D = 128
E = 320000
CHUNK = 128
NCHUNKS = 2560
EPAD = NCHUNKS * CHUNK - E
NREAL = E // CHUNK
NC, NS = 2, 16
NW = NC * NS
CPT = NCHUNKS // NW
PASSES = 2
CPP = CPT // PASSES
NPAD = 10240
RPS = NPAD // NS
ZROWS = 16
LANES = 16


def _matmul_body(h_ref, w_ref, norm_ref, x_ref):
    x_ref[...] = (
        jnp.dot(h_ref[...], w_ref[...], preferred_element_type=jnp.float32)
        * norm_ref[...]
    )


def _xw_norm(h, W, norm):
    mb = 2000
    return pl.pallas_call(
        _matmul_body,
        grid=(N // mb,),
        in_specs=[
            pl.BlockSpec((mb, D), lambda i: (i, 0)),
            pl.BlockSpec((D, D), lambda i: (0, 0)),
            pl.BlockSpec((mb, 1), lambda i: (i, 0)),
        ],
        out_specs=pl.BlockSpec((mb, D), lambda i: (i, 0)),
        out_shape=jax.ShapeDtypeStruct((N, D), jnp.float32),
    )(h, W, norm)


def _sc_body(x_hbm, edge_hbm, out_hbm,
             sidx, didx, rows0, rows1, zbuf, acc,
             sem0, sem1, dsem):
    c = lax.axis_index("c")
    s = lax.axis_index("s")
    wid = c * NS + s

    zeros = jnp.zeros((LANES,), jnp.float32)

    def zrow(r, carry):
        for k in range(D // LANES):
            zbuf[r, pl.ds(k * LANES, LANES)] = zeros
        return carry

    lax.fori_loop(0, ZROWS, zrow, 0)

    def zcopy(i, carry):
        pltpu.sync_copy(zbuf, acc.at[pl.ds(s * RPS + i * ZROWS, ZROWS)])
        return carry

    rows = (rows0, rows1)
    sems = (sem0, sem1)
    ebase = wid * CPT

    def sidx_at(j):
        return sidx.at[pl.ds(j * CHUNK, CHUNK)]

    def stage_idx(base):
        sbase = jnp.minimum(base, NREAL - CPP)
        pltpu.sync_copy(
            edge_hbm.at[0, pl.ds(sbase * CHUNK, CPP * CHUNK)], sidx)
        for j in range(CPP):
            pltpu.async_copy(
                edge_hbm.at[1, pl.ds((sbase + j) * CHUNK, CHUNK)],
                didx.at[j], dsem)
        for j in range(CPP):
            pltpu.make_async_copy(
                edge_hbm.at[1, pl.ds((sbase + j) * CHUNK, CHUNK)],
                didx.at[j], dsem).wait()
        return base - sbase

    def prime(base, soff):
        for b in range(2):
            @pl.when(base + b < NREAL)
            def _():
                pltpu.async_copy(x_hbm.at[sidx_at(b + soff)], rows[b], sems[b])

    soff = stage_idx(ebase)
    prime(ebase, soff)

    lax.fori_loop(0, RPS // ZROWS, zcopy, 0)
    plsc.subcore_barrier()

    for p in range(PASSES):
        base = ebase + p * CPP
        if p > 0:
            soff = stage_idx(base)
            prime(base, soff)

        def body(g, carry):
            for b in range(2):
                j = g * 2 + b

                @pl.when(base + j < NREAL)
                def _():
                    pltpu.make_async_copy(
                        x_hbm.at[sidx_at(j + soff)], rows[b], sems[b]).wait()
                    pltpu.sync_copy(
                        rows[b], acc.at[didx.at[j + soff]], add=True)

                    @pl.when((j + 2 < CPP) & (base + j + 2 < NREAL))
                    def _():
                        pltpu.async_copy(
                            x_hbm.at[sidx_at(j + 2 + soff)], rows[b], sems[b])

            return carry

        lax.fori_loop(0, CPP // 2, body, 0)

    plsc.subcore_barrier()
    pltpu.sync_copy(
        acc.at[pl.ds(s * RPS, RPS)],
        out_hbm.at[c].at[pl.ds(s * RPS, RPS)],
    )


def _sc_segment_sum(x, edge_index):
    mesh = plsc.VectorSubcoreMesh(core_axis_name="c", subcore_axis_name="s")
    f = functools.partial(
        pl.kernel,
        out_type=jax.ShapeDtypeStruct((NC, NPAD, D), jnp.float32),
        mesh=mesh,
        scratch_types=[
            pltpu.VMEM((CPP * CHUNK,), jnp.int32),
            pltpu.VMEM((CPP, CHUNK), jnp.int32),
            pltpu.VMEM((CHUNK, D), jnp.float32),
            pltpu.VMEM((CHUNK, D), jnp.float32),
            pltpu.VMEM((ZROWS, D), jnp.float32),
            pltpu.VMEM_SHARED((NPAD, D), jnp.float32),
            pltpu.SemaphoreType.DMA,
            pltpu.SemaphoreType.DMA,
            pltpu.SemaphoreType.DMA,
        ],
    )(_sc_body)
    return f(x, edge_index)


def _combine_body(p_ref, norm_ref, b_ref, o_ref):
    o_ref[...] = jnp.maximum(
        (p_ref[0] + p_ref[1]) * norm_ref[...] + b_ref[...], 0.0
    )


def _combine(partials, norm, b2d):
    mb = 2000
    return pl.pallas_call(
        _combine_body,
        grid=(N // mb,),
        in_specs=[
            pl.BlockSpec((NC, mb, D), lambda i: (0, i, 0)),
            pl.BlockSpec((mb, 1), lambda i: (i, 0)),
            pl.BlockSpec((1, D), lambda i: (0, 0)),
        ],
        out_specs=pl.BlockSpec((mb, D), lambda i: (i, 0)),
        out_shape=jax.ShapeDtypeStruct((N, D), jnp.float32),
    )(partials, norm, b2d)


def kernel(h, edge_index, norm, W, b):
    x = _xw_norm(h, W, norm)
    partials = _sc_segment_sum(x, edge_index)
    return _combine(partials, norm, b.reshape(1, D))


# speedup vs baseline: 3.6152x; 1.0217x over previous
import functools

import jax
import jax.numpy as jnp
from jax import lax
from jax.experimental import pallas as pl
from jax.experimental.pallas import tpu as pltpu
from jax.experimental.pallas import tpu_sc as plsc

N = 10000
D = 128
E = 320000
CHUNK = 128
NCHUNKS = 2560
EPAD = NCHUNKS * CHUNK - E
NREAL = E // CHUNK
NC, NS = 2, 16
NW = NC * NS
CPT = NCHUNKS // NW
PASSES = 2
CPP = CPT // PASSES
NPAD = 10240
RPS = NPAD // NS
ZROWS = 16
LANES = 16


def _matmul_body(h_ref, w_ref, norm_ref, x_ref):
    x_ref[...] = (
        jnp.dot(h_ref[...], w_ref[...], preferred_element_type=jnp.float32)
        * norm_ref[...]
    )


def _xw_norm(h, W, norm):
    mb = 5000
    return pl.pallas_call(
        _matmul_body,
        grid=(N // mb,),
        in_specs=[
            pl.BlockSpec((mb, D), lambda i: (i, 0)),
            pl.BlockSpec((D, D), lambda i: (0, 0)),
            pl.BlockSpec((mb, 1), lambda i: (i, 0)),
        ],
        out_specs=pl.BlockSpec((mb, D), lambda i: (i, 0)),
        out_shape=jax.ShapeDtypeStruct((N, D), jnp.float32),
    )(h, W, norm)


def _sc_body(x_hbm, edge_hbm, out_hbm,
             sidx, didx, rows0, rows1, zbuf, acc,
             sem0, sem1, dsem):
    c = lax.axis_index("c")
    s = lax.axis_index("s")
    wid = c * NS + s

    zeros = jnp.zeros((LANES,), jnp.float32)

    def zrow(r, carry):
        for k in range(D // LANES):
            zbuf[r, pl.ds(k * LANES, LANES)] = zeros
        return carry

    lax.fori_loop(0, ZROWS, zrow, 0)

    def zcopy(i, carry):
        pltpu.sync_copy(zbuf, acc.at[pl.ds(s * RPS + i * ZROWS, ZROWS)])
        return carry

    rows = (rows0, rows1)
    sems = (sem0, sem1)
    ebase = wid * CPT

    def sidx_at(j):
        return sidx.at[pl.ds(j * CHUNK, CHUNK)]

    def stage_idx(base):
        sbase = jnp.minimum(base, NREAL - CPP)
        pltpu.sync_copy(
            edge_hbm.at[0, pl.ds(sbase * CHUNK, CPP * CHUNK)], sidx)
        for j in range(CPP):
            pltpu.async_copy(
                edge_hbm.at[1, pl.ds((sbase + j) * CHUNK, CHUNK)],
                didx.at[j], dsem)
        for j in range(CPP):
            pltpu.make_async_copy(
                edge_hbm.at[1, pl.ds((sbase + j) * CHUNK, CHUNK)],
                didx.at[j], dsem).wait()
        return base - sbase

    def prime(base, soff):
        for b in range(2):
            @pl.when(base + b < NREAL)
            def _():
                pltpu.async_copy(x_hbm.at[sidx_at(b + soff)], rows[b], sems[b])

    soff = stage_idx(ebase)
    prime(ebase, soff)

    lax.fori_loop(0, RPS // ZROWS, zcopy, 0)
    plsc.subcore_barrier()

    for p in range(PASSES):
        base = ebase + p * CPP
        if p > 0:
            soff = stage_idx(base)
            prime(base, soff)

        def body(g, carry):
            for b in range(2):
                j = g * 2 + b

                @pl.when(base + j < NREAL)
                def _():
                    pltpu.make_async_copy(
                        x_hbm.at[sidx_at(j + soff)], rows[b], sems[b]).wait()
                    pltpu.sync_copy(
                        rows[b], acc.at[didx.at[j + soff]], add=True)

                    @pl.when((j + 2 < CPP) & (base + j + 2 < NREAL))
                    def _():
                        pltpu.async_copy(
                            x_hbm.at[sidx_at(j + 2 + soff)], rows[b], sems[b])

            return carry

        lax.fori_loop(0, CPP // 2, body, 0)

    plsc.subcore_barrier()
    pltpu.sync_copy(
        acc.at[pl.ds(s * RPS, RPS)],
        out_hbm.at[c].at[pl.ds(s * RPS, RPS)],
    )


def _sc_segment_sum(x, edge_index):
    mesh = plsc.VectorSubcoreMesh(core_axis_name="c", subcore_axis_name="s")
    f = functools.partial(
        pl.kernel,
        out_type=jax.ShapeDtypeStruct((NC, NPAD, D), jnp.float32),
        mesh=mesh,
        scratch_types=[
            pltpu.VMEM((CPP * CHUNK,), jnp.int32),
            pltpu.VMEM((CPP, CHUNK), jnp.int32),
            pltpu.VMEM((CHUNK, D), jnp.float32),
            pltpu.VMEM((CHUNK, D), jnp.float32),
            pltpu.VMEM((ZROWS, D), jnp.float32),
            pltpu.VMEM_SHARED((NPAD, D), jnp.float32),
            pltpu.SemaphoreType.DMA,
            pltpu.SemaphoreType.DMA,
            pltpu.SemaphoreType.DMA,
        ],
    )(_sc_body)
    return f(x, edge_index)


def _combine_body(p_ref, norm_ref, b_ref, o_ref):
    o_ref[...] = jnp.maximum(
        (p_ref[0] + p_ref[1]) * norm_ref[...] + b_ref[...], 0.0
    )


def _combine(partials, norm, b2d):
    mb = 5000
    return pl.pallas_call(
        _combine_body,
        grid=(N // mb,),
        in_specs=[
            pl.BlockSpec((NC, mb, D), lambda i: (0, i, 0)),
            pl.BlockSpec((mb, 1), lambda i: (i, 0)),
            pl.BlockSpec((1, D), lambda i: (0, 0)),
        ],
        out_specs=pl.BlockSpec((mb, D), lambda i: (i, 0)),
        out_shape=jax.ShapeDtypeStruct((N, D), jnp.float32),
    )(partials, norm, b2d)


def kernel(h, edge_index, norm, W, b):
    x = _xw_norm(h, W, norm)
    partials = _sc_segment_sum(x, edge_index)
    return _combine(partials, norm, b.reshape(1, D))
